# baseline (speedup 1.0000x reference)
import functools

import jax
import jax.numpy as jnp
from jax import lax
from jax.experimental import pallas as pl
from jax.experimental.pallas import tpu as pltpu
from jax.experimental.pallas import tpu_sc as plsc

RES = 256
DT = 1e-4
E_MOD = 10000.0
NU = 0.2
RHO = 1.0
MU0 = E_MOD / (2.0 * (1.0 + NU))
LAM0 = E_MOD * NU / ((1.0 + NU) * (1.0 - 2.0 * NU))
DX = 1.0 / RES
VOL = DX * DX * 0.5
RV = RHO * VOL
KS = -DT * VOL * (4.0 / DX ** 2)
N_P = 524288
N_TILES = 32
P_TILE = N_P // N_TILES
CH = 1024
N_CHUNK = P_TILE // CH
GROUPS = CH // 16
HALF = (RES * RES) // 2

_mesh = plsc.VectorSubcoreMesh(core_axis_name="c", subcore_axis_name="s",
                               num_cores=2, num_subcores=16)


def build_p2g(n_p, n_tiles, ch, interpret=False):
  p_tile = n_p // n_tiles
  n_chunk = p_tile // ch
  groups = ch // 16

  @functools.partial(
      pl.kernel,
      out_type=jax.ShapeDtypeStruct((6, n_tiles, 1, HALF), jnp.float32),
      mesh=_mesh,
      compiler_params=pltpu.CompilerParams(needs_layout_passes=False),
      interpret=interpret,
      scratch_types=[
          [pltpu.VMEM((ch,), jnp.float32) for _ in range(12)],
          [pltpu.VMEM((ch,), jnp.float32) for _ in range(12)],
          pltpu.VMEM((HALF,), jnp.float32),
          pltpu.VMEM((HALF,), jnp.float32),
          pltpu.VMEM((HALF,), jnp.float32),
          pltpu.SemaphoreType.DMA,
          pltpu.SemaphoreType.DMA,
      ],
  )
  def _p2g_sc(px_h, py_h, vx_h, vy_h, f00_h, f01_h, f10_h, f11_h,
              c00_h, c01_h, c10_h, c11_h, out_hbm,
              buf_a, buf_b, mass_v, mx_v, my_v, sem_a, sem_b):
    CH = ch
    P_TILE = p_tile
    N_CHUNK = n_chunk
    GROUPS = groups
    inputs_h = (px_h, py_h, vx_h, vy_h, f00_h, f01_h, f10_h, f11_h,
                c00_h, c01_h, c10_h, c11_h)
    bufs = (buf_a, buf_b)
    sems = (sem_a, sem_b)
    wid = lax.axis_index("s") * 2 + lax.axis_index("c")
    pbase = wid * P_TILE
    zf = jnp.zeros((16,), jnp.float32)

    def fire(k, which):
        src = pl.ds(pbase + k * CH, CH)
        for h, v in zip(inputs_h, bufs[which]):
            pltpu.async_copy(h.at[src], v, sems[which])

    def wait(k, which):
        src = pl.ds(pbase + k * CH, CH)
        for h, v in zip(inputs_h, bufs[which]):
            pltpu.make_async_copy(h.at[src], v, sems[which]).wait()

    def make_group_body(comp_v, half):
        def group_body(g):
                rows = g * 16 + lax.broadcasted_iota(jnp.int32, (16,), 0)
                (px, py, vx, vy, f00, f01, f10, f11,
                 c00, c01, c10, c11) = [
                    plsc.load_gather(r, [rows]) for r in comp_v]

                det = f00 * f11 - f01 * f10
                rdet = 1.0 / det
                j_cl = jnp.maximum(det, 1e-8)
                nrdet = -rdet
                ft00 = f11 * rdet
                ft01 = f10 * nrdet
                ft10 = f01 * nrdet
                ft11 = f00 * rdet
                b = LAM0 * (j_cl - 1.0) - MU0
                p00 = MU0 * f00 + b * ft00
                p01 = MU0 * f01 + b * ft01
                p10 = MU0 * f10 + b * ft10
                p11 = MU0 * f11 + b * ft11
                a00 = KS * (p00 * f00 + p01 * f01) + RV * c00
                a01 = KS * (p00 * f10 + p01 * f11) + RV * c01
                a10 = KS * (p10 * f00 + p11 * f01) + RV * c10
                a11 = KS * (p10 * f10 + p11 * f11) + RV * c11

                xpx = px * float(RES) - 0.5
                xpy = py * float(RES) - 0.5
                bx = xpx.astype(jnp.int32)
                by = xpy.astype(jnp.int32)
                fxx = xpx - bx.astype(jnp.float32)
                fxy = xpy - by.astype(jnp.float32)

                tx0 = 1.5 - fxx
                tx1 = fxx - 1.0
                tx2 = fxx - 0.5
                ty0 = 1.5 - fxy
                ty1 = fxy - 1.0
                ty2 = fxy - 0.5
                wx = (0.5 * (tx0 * tx0),
                      0.75 - tx1 * tx1,
                      0.5 * (tx2 * tx2))
                wy = (0.5 * (ty0 * ty0),
                      0.75 - ty1 * ty1,
                      0.5 * (ty2 * ty2))
                wxm = tuple(w * RV for w in wx)

                mvx = RV * vx
                mvy = RV * vy
                dpx = tuple((float(i) - fxx) * DX for i in range(3))
                dpy = tuple((float(j) - fxy) * DX for j in range(3))
                xu = tuple(mvx + a00 * d for d in dpx)
                yv = tuple(mvy + a10 * d for d in dpx)
                v_j = tuple(a01 * d for d in dpy)
                s_j = tuple(a11 * d for d in dpy)

                nxs0 = tuple(
                    jnp.clip(bx + i, 0, RES - 1) << 8 for i in range(3))
                nys = tuple(
                    jnp.clip(by + j, 0, RES - 1) for j in range(3))
                if half == 0:
                    masks = tuple(n < HALF for n in nxs0)
                    nxs = nxs0
                else:
                    masks = tuple(n >= HALF for n in nxs0)
                    nxs = tuple(n - HALF for n in nxs0)

                for i in range(3):
                    m = masks[i]
                    for j in range(3):
                        idx = nxs[i] + nys[j]
                        ww = wx[i] * wy[j]
                        mass_c = wxm[i] * wy[j]
                        momx_c = ww * (xu[i] + v_j[j])
                        momy_c = ww * (yv[i] + s_j[j])
                        plsc.addupdate_scatter(mass_v, [idx], mass_c, mask=m)
                        plsc.addupdate_scatter(mx_v, [idx], momx_c, mask=m)
                        plsc.addupdate_scatter(my_v, [idx], momy_c, mask=m)
        return group_body

    for half in range(2):
        def zero_body(i, carry):
            sl = pl.ds(i * 16, 16)
            mass_v[sl] = zf
            mx_v[sl] = zf
            my_v[sl] = zf
            return carry
        lax.fori_loop(0, HALF // 16, zero_body, 0)

        fire(0, 0)

        def pair_body(t, carry):
            k0 = 2 * t
            k1 = k0 + 1
            wait(k0, 0)
            fire(k1, 1)
            plsc.parallel_loop(0, GROUPS, unroll=4)(
                make_group_body(bufs[0], half))
            wait(k1, 1)

            @pl.when(k1 + 1 < N_CHUNK)
            def _prefetch():
                fire(k1 + 1, 0)

            plsc.parallel_loop(0, GROUPS, unroll=4)(
                make_group_body(bufs[1], half))
            return carry

        lax.fori_loop(0, N_CHUNK // 2, pair_body, 0)

        pltpu.sync_copy(mass_v, out_hbm.at[3 * half + 0, wid, 0])
        pltpu.sync_copy(mx_v, out_hbm.at[3 * half + 1, wid, 0])
        pltpu.sync_copy(my_v, out_hbm.at[3 * half + 2, wid, 0])

  return _p2g_sc


_p2g_sc = build_p2g(N_P, N_TILES, CH)


def _red_body(x_ref, o_ref):
    o_ref[...] = jnp.sum(x_ref[...], axis=1)


_reduce_tc = pl.pallas_call(
    _red_body,
    grid=(6,),
    in_specs=[pl.BlockSpec((1, N_TILES, 1, HALF), lambda k: (k, 0, 0, 0))],
    out_specs=pl.BlockSpec((1, 1, HALF), lambda k: (k, 0, 0)),
    out_shape=jax.ShapeDtypeStruct((6, 1, HALF), jnp.float32),
)


def kernel(pos, vel, F, C, Jp):
    del Jp
    parts = _p2g_sc(
        pos[:, 0], pos[:, 1], vel[:, 0], vel[:, 1],
        F[:, 0, 0], F[:, 0, 1], F[:, 1, 0], F[:, 1, 1],
        C[:, 0, 0], C[:, 0, 1], C[:, 1, 0], C[:, 1, 1])
    red = _reduce_tc(parts).reshape(2, 3, HALF)
    grid_mass = jnp.concatenate([red[0, 0], red[1, 0]])
    momx = jnp.concatenate([red[0, 1], red[1, 1]])
    momy = jnp.concatenate([red[0, 2], red[1, 2]])
    grid_mom = jnp.stack([momx, momy], axis=-1)
    return grid_mom, grid_mass


# speedup vs baseline: 1.5954x; 1.5954x over previous
import functools

import jax
import jax.numpy as jnp
from jax import lax
from jax.experimental import pallas as pl
from jax.experimental.pallas import tpu as pltpu
from jax.experimental.pallas import tpu_sc as plsc

RES = 256
DT = 1e-4
E_MOD = 10000.0
NU = 0.2
RHO = 1.0
MU0 = E_MOD / (2.0 * (1.0 + NU))
LAM0 = E_MOD * NU / ((1.0 + NU) * (1.0 - 2.0 * NU))
DX = 1.0 / RES
VOL = DX * DX * 0.5
RV = RHO * VOL
KS = -DT * VOL * (4.0 / DX ** 2)
N_P = 524288
N_TILES = 32
P_TILE = N_P // N_TILES
CH = 1024
N_CHUNK = P_TILE // CH
GROUPS = CH // 16
HALF = (RES * RES) // 2

_mesh = plsc.VectorSubcoreMesh(core_axis_name="c", subcore_axis_name="s",
                               num_cores=2, num_subcores=16)


def build_p2g(n_p, n_tiles, ch, interpret=False):
  p_tile = n_p // n_tiles
  n_chunk = p_tile // ch
  groups = ch // 16

  @functools.partial(
      pl.kernel,
      out_type=jax.ShapeDtypeStruct((6, n_tiles, 1, HALF), jnp.float32),
      mesh=_mesh,
      compiler_params=pltpu.CompilerParams(needs_layout_passes=False),
      interpret=interpret,
      scratch_types=[
          [pltpu.VMEM((2 * ch,), jnp.float32) for _ in range(6)],
          [pltpu.VMEM((2 * ch,), jnp.float32) for _ in range(6)],
          pltpu.VMEM((HALF,), jnp.float32),
          pltpu.VMEM((HALF,), jnp.float32),
          pltpu.VMEM((HALF,), jnp.float32),
          pltpu.SemaphoreType.DMA,
          pltpu.SemaphoreType.DMA,
      ],
  )
  def _p2g_sc(pxy_h, vxy_h, f0_h, f1_h, c0_h, c1_h, out_hbm,
              buf_a, buf_b, mass_v, mx_v, my_v, sem_a, sem_b):
    CH = ch
    P_TILE = p_tile
    N_CHUNK = n_chunk
    GROUPS = groups
    inputs_h = (pxy_h, vxy_h, f0_h, f1_h, c0_h, c1_h)
    bufs = (buf_a, buf_b)
    sems = (sem_a, sem_b)
    wid = lax.axis_index("s") * 2 + lax.axis_index("c")
    pbase = wid * P_TILE
    zf = jnp.zeros((16,), jnp.float32)

    def fire(k, which):
        src = pl.ds((pbase + k * CH) * 2, CH * 2)
        for h, v in zip(inputs_h, bufs[which]):
            pltpu.async_copy(h.at[src], v, sems[which])

    def wait(k, which):
        src = pl.ds((pbase + k * CH) * 2, CH * 2)
        for h, v in zip(inputs_h, bufs[which]):
            pltpu.make_async_copy(h.at[src], v, sems[which]).wait()

    def make_group_body(comp_v, half):
        def group_body(g):
                base = (g // 8) * 256 + (g % 8) * 16
                r0 = base + lax.broadcasted_iota(jnp.int32, (16,), 0)
                r1 = r0 + 128
                pxy_b, vxy_b, f0_b, f1_b, c0_b, c1_b = comp_v
                px = plsc.load_gather(pxy_b, [r0])
                py = plsc.load_gather(pxy_b, [r1])
                vx = plsc.load_gather(vxy_b, [r0])
                vy = plsc.load_gather(vxy_b, [r1])
                f00 = plsc.load_gather(f0_b, [r0])
                f01 = plsc.load_gather(f0_b, [r1])
                f10 = plsc.load_gather(f1_b, [r0])
                f11 = plsc.load_gather(f1_b, [r1])
                c00 = plsc.load_gather(c0_b, [r0])
                c01 = plsc.load_gather(c0_b, [r1])
                c10 = plsc.load_gather(c1_b, [r0])
                c11 = plsc.load_gather(c1_b, [r1])

                det = f00 * f11 - f01 * f10
                rdet = 1.0 / det
                j_cl = jnp.maximum(det, 1e-8)
                nrdet = -rdet
                ft00 = f11 * rdet
                ft01 = f10 * nrdet
                ft10 = f01 * nrdet
                ft11 = f00 * rdet
                b = LAM0 * (j_cl - 1.0) - MU0
                p00 = MU0 * f00 + b * ft00
                p01 = MU0 * f01 + b * ft01
                p10 = MU0 * f10 + b * ft10
                p11 = MU0 * f11 + b * ft11
                a00 = KS * (p00 * f00 + p01 * f01) + RV * c00
                a01 = KS * (p00 * f10 + p01 * f11) + RV * c01
                a10 = KS * (p10 * f00 + p11 * f01) + RV * c10
                a11 = KS * (p10 * f10 + p11 * f11) + RV * c11

                xpx = px * float(RES) - 0.5
                xpy = py * float(RES) - 0.5
                bx = xpx.astype(jnp.int32)
                by = xpy.astype(jnp.int32)
                fxx = xpx - bx.astype(jnp.float32)
                fxy = xpy - by.astype(jnp.float32)

                tx0 = 1.5 - fxx
                tx1 = fxx - 1.0
                tx2 = fxx - 0.5
                ty0 = 1.5 - fxy
                ty1 = fxy - 1.0
                ty2 = fxy - 0.5
                wx = (0.5 * (tx0 * tx0),
                      0.75 - tx1 * tx1,
                      0.5 * (tx2 * tx2))
                wy = (0.5 * (ty0 * ty0),
                      0.75 - ty1 * ty1,
                      0.5 * (ty2 * ty2))
                wxm = tuple(w * RV for w in wx)

                mvx = RV * vx
                mvy = RV * vy
                dpx = tuple((float(i) - fxx) * DX for i in range(3))
                dpy = tuple((float(j) - fxy) * DX for j in range(3))
                xu = tuple(mvx + a00 * d for d in dpx)
                yv = tuple(mvy + a10 * d for d in dpx)
                v_j = tuple(a01 * d for d in dpy)
                s_j = tuple(a11 * d for d in dpy)

                nxs0 = tuple(
                    jnp.clip(bx + i, 0, RES - 1) << 8 for i in range(3))
                nys = tuple(
                    jnp.clip(by + j, 0, RES - 1) for j in range(3))
                if half == 0:
                    masks = tuple(n < HALF for n in nxs0)
                    nxs = nxs0
                else:
                    masks = tuple(n >= HALF for n in nxs0)
                    nxs = tuple(n - HALF for n in nxs0)

                for i in range(3):
                    m = masks[i]
                    for j in range(3):
                        idx = nxs[i] + nys[j]
                        ww = wx[i] * wy[j]
                        mass_c = wxm[i] * wy[j]
                        momx_c = ww * (xu[i] + v_j[j])
                        momy_c = ww * (yv[i] + s_j[j])
                        plsc.addupdate_scatter(mass_v, [idx], mass_c, mask=m)
                        plsc.addupdate_scatter(mx_v, [idx], momx_c, mask=m)
                        plsc.addupdate_scatter(my_v, [idx], momy_c, mask=m)
        return group_body

    for half in range(2):
        def zero_body(i, carry):
            sl = pl.ds(i * 16, 16)
            mass_v[sl] = zf
            mx_v[sl] = zf
            my_v[sl] = zf
            return carry
        lax.fori_loop(0, HALF // 16, zero_body, 0)

        fire(0, 0)

        def pair_body(t, carry):
            k0 = 2 * t
            k1 = k0 + 1
            wait(k0, 0)
            fire(k1, 1)
            plsc.parallel_loop(0, GROUPS, unroll=2)(
                make_group_body(bufs[0], half))
            wait(k1, 1)

            @pl.when(k1 + 1 < N_CHUNK)
            def _prefetch():
                fire(k1 + 1, 0)

            plsc.parallel_loop(0, GROUPS, unroll=2)(
                make_group_body(bufs[1], half))
            return carry

        lax.fori_loop(0, N_CHUNK // 2, pair_body, 0)

        pltpu.sync_copy(mass_v, out_hbm.at[3 * half + 0, wid, 0])
        pltpu.sync_copy(mx_v, out_hbm.at[3 * half + 1, wid, 0])
        pltpu.sync_copy(my_v, out_hbm.at[3 * half + 2, wid, 0])

  return _p2g_sc


_p2g_sc = build_p2g(N_P, N_TILES, CH)


def _red_body(x_ref, o_ref):
    o_ref[...] = jnp.sum(x_ref[...], axis=1)


_reduce_tc = pl.pallas_call(
    _red_body,
    grid=(6,),
    in_specs=[pl.BlockSpec((1, N_TILES, 1, HALF), lambda k: (k, 0, 0, 0))],
    out_specs=pl.BlockSpec((1, 1, HALF), lambda k: (k, 0, 0)),
    out_shape=jax.ShapeDtypeStruct((6, 1, HALF), jnp.float32),
)


def _iview(u):
    return u.reshape(-1, 128, 2).transpose(0, 2, 1).reshape(-1)


def kernel(pos, vel, F, C, Jp):
    del Jp
    parts = _p2g_sc(
        _iview(pos), _iview(vel),
        _iview(F[:, 0, :]), _iview(F[:, 1, :]),
        _iview(C[:, 0, :]), _iview(C[:, 1, :]))
    red = _reduce_tc(parts).reshape(2, 3, HALF)
    grid_mass = jnp.concatenate([red[0, 0], red[1, 0]])
    momx = jnp.concatenate([red[0, 1], red[1, 1]])
    momy = jnp.concatenate([red[0, 2], red[1, 2]])
    grid_mom = jnp.stack([momx, momy], axis=-1)
    return grid_mom, grid_mass


# speedup vs baseline: 1.6592x; 1.0400x over previous
import functools

import jax
import jax.numpy as jnp
from jax import lax
from jax.experimental import pallas as pl
from jax.experimental.pallas import tpu as pltpu
from jax.experimental.pallas import tpu_sc as plsc

RES = 256
DT = 1e-4
E_MOD = 10000.0
NU = 0.2
RHO = 1.0
MU0 = E_MOD / (2.0 * (1.0 + NU))
LAM0 = E_MOD * NU / ((1.0 + NU) * (1.0 - 2.0 * NU))
DX = 1.0 / RES
VOL = DX * DX * 0.5
RV = RHO * VOL
KS = -DT * VOL * (4.0 / DX ** 2)
N_P = 524288
N_TILES = 32
P_TILE = N_P // N_TILES
CH = 1024
N_CHUNK = P_TILE // CH
GROUPS = CH // 16
HALF = (RES * RES) // 2

_mesh = plsc.VectorSubcoreMesh(core_axis_name="c", subcore_axis_name="s",
                               num_cores=2, num_subcores=16)


def build_p2g(n_p, n_tiles, ch, interpret=False):
  p_tile = n_p // n_tiles
  n_chunk = p_tile // ch
  groups = ch // 16

  @functools.partial(
      pl.kernel,
      out_type=jax.ShapeDtypeStruct((6, n_tiles, 1, HALF), jnp.float32),
      mesh=_mesh,
      compiler_params=pltpu.CompilerParams(needs_layout_passes=False),
      interpret=interpret,
      scratch_types=[
          [pltpu.VMEM((2 * ch,), jnp.float32) for _ in range(6)],
          [pltpu.VMEM((2 * ch,), jnp.float32) for _ in range(6)],
          pltpu.VMEM((HALF,), jnp.float32),
          pltpu.VMEM((HALF,), jnp.float32),
          pltpu.VMEM((HALF,), jnp.float32),
          pltpu.SemaphoreType.DMA,
          pltpu.SemaphoreType.DMA,
      ],
  )
  def _p2g_sc(pxy_h, vxy_h, f_h, c_h, out_hbm,
              buf_a, buf_b, mass_v, mx_v, my_v, sem_a, sem_b):
    CH = ch
    P_TILE = p_tile
    N_CHUNK = n_chunk
    GROUPS = groups
    n2 = n_p * 2
    bufs = (buf_a, buf_b)
    sems = (sem_a, sem_b)
    wid = lax.axis_index("s") * 2 + lax.axis_index("c")
    pbase = wid * P_TILE
    zf = jnp.zeros((16,), jnp.float32)

    def srcs(k):
        w = (pbase + k * CH) * 2
        return ((pxy_h, w), (vxy_h, w), (f_h, w), (f_h, n2 + w),
                (c_h, w), (c_h, n2 + w))

    def fire(k, which):
        for (h, off), v in zip(srcs(k), bufs[which]):
            pltpu.async_copy(h.at[pl.ds(off, CH * 2)], v, sems[which])

    def wait(k, which):
        for (h, off), v in zip(srcs(k), bufs[which]):
            pltpu.make_async_copy(h.at[pl.ds(off, CH * 2)], v,
                                  sems[which]).wait()

    def make_group_body(comp_v, half):
        def group_body(g):
                base = (g // 8) * 256 + (g % 8) * 16
                r0 = base + lax.broadcasted_iota(jnp.int32, (16,), 0)
                r1 = r0 + 128
                pxy_b, vxy_b, f0_b, f1_b, c0_b, c1_b = comp_v
                px = plsc.load_gather(pxy_b, [r0])
                py = plsc.load_gather(pxy_b, [r1])
                vx = plsc.load_gather(vxy_b, [r0])
                vy = plsc.load_gather(vxy_b, [r1])
                f00 = plsc.load_gather(f0_b, [r0])
                f01 = plsc.load_gather(f0_b, [r1])
                f10 = plsc.load_gather(f1_b, [r0])
                f11 = plsc.load_gather(f1_b, [r1])
                c00 = plsc.load_gather(c0_b, [r0])
                c01 = plsc.load_gather(c0_b, [r1])
                c10 = plsc.load_gather(c1_b, [r0])
                c11 = plsc.load_gather(c1_b, [r1])

                det = f00 * f11 - f01 * f10
                rdet = 1.0 / det
                j_cl = jnp.maximum(det, 1e-8)
                nrdet = -rdet
                ft00 = f11 * rdet
                ft01 = f10 * nrdet
                ft10 = f01 * nrdet
                ft11 = f00 * rdet
                b = LAM0 * (j_cl - 1.0) - MU0
                p00 = MU0 * f00 + b * ft00
                p01 = MU0 * f01 + b * ft01
                p10 = MU0 * f10 + b * ft10
                p11 = MU0 * f11 + b * ft11
                a00 = KS * (p00 * f00 + p01 * f01) + RV * c00
                a01 = KS * (p00 * f10 + p01 * f11) + RV * c01
                a10 = KS * (p10 * f00 + p11 * f01) + RV * c10
                a11 = KS * (p10 * f10 + p11 * f11) + RV * c11

                xpx = px * float(RES) - 0.5
                xpy = py * float(RES) - 0.5
                bx = xpx.astype(jnp.int32)
                by = xpy.astype(jnp.int32)
                fxx = xpx - bx.astype(jnp.float32)
                fxy = xpy - by.astype(jnp.float32)

                tx0 = 1.5 - fxx
                tx1 = fxx - 1.0
                tx2 = fxx - 0.5
                ty0 = 1.5 - fxy
                ty1 = fxy - 1.0
                ty2 = fxy - 0.5
                wx = (0.5 * (tx0 * tx0),
                      0.75 - tx1 * tx1,
                      0.5 * (tx2 * tx2))
                wy = (0.5 * (ty0 * ty0),
                      0.75 - ty1 * ty1,
                      0.5 * (ty2 * ty2))
                wxm = tuple(w * RV for w in wx)

                mvx = RV * vx
                mvy = RV * vy
                dpx = tuple((float(i) - fxx) * DX for i in range(3))
                dpy = tuple((float(j) - fxy) * DX for j in range(3))
                xu = tuple(mvx + a00 * d for d in dpx)
                yv = tuple(mvy + a10 * d for d in dpx)
                v_j = tuple(a01 * d for d in dpy)
                s_j = tuple(a11 * d for d in dpy)

                nxs0 = tuple(
                    jnp.clip(bx + i, 0, RES - 1) << 8 for i in range(3))
                nys = tuple(
                    jnp.clip(by + j, 0, RES - 1) for j in range(3))
                if half == 0:
                    masks = tuple(n < HALF for n in nxs0)
                    nxs = nxs0
                else:
                    masks = tuple(n >= HALF for n in nxs0)
                    nxs = tuple(n - HALF for n in nxs0)

                for i in range(3):
                    m = masks[i]
                    for j in range(3):
                        idx = nxs[i] + nys[j]
                        ww = wx[i] * wy[j]
                        mass_c = wxm[i] * wy[j]
                        momx_c = ww * (xu[i] + v_j[j])
                        momy_c = ww * (yv[i] + s_j[j])
                        plsc.addupdate_scatter(mass_v, [idx], mass_c, mask=m)
                        plsc.addupdate_scatter(mx_v, [idx], momx_c, mask=m)
                        plsc.addupdate_scatter(my_v, [idx], momy_c, mask=m)
        return group_body

    for half in range(2):
        def zero_body(i, carry):
            sl = pl.ds(i * 16, 16)
            mass_v[sl] = zf
            mx_v[sl] = zf
            my_v[sl] = zf
            return carry
        lax.fori_loop(0, HALF // 16, zero_body, 0)

        fire(0, 0)

        def pair_body(t, carry):
            k0 = 2 * t
            k1 = k0 + 1
            wait(k0, 0)
            fire(k1, 1)
            plsc.parallel_loop(0, GROUPS, unroll=2)(
                make_group_body(bufs[0], half))
            wait(k1, 1)

            @pl.when(k1 + 1 < N_CHUNK)
            def _prefetch():
                fire(k1 + 1, 0)

            plsc.parallel_loop(0, GROUPS, unroll=2)(
                make_group_body(bufs[1], half))
            return carry

        lax.fori_loop(0, N_CHUNK // 2, pair_body, 0)

        pltpu.sync_copy(mass_v, out_hbm.at[3 * half + 0, wid, 0])
        pltpu.sync_copy(mx_v, out_hbm.at[3 * half + 1, wid, 0])
        pltpu.sync_copy(my_v, out_hbm.at[3 * half + 2, wid, 0])

  return _p2g_sc


_p2g_sc = build_p2g(N_P, N_TILES, CH)


def _red_body(x_ref, o_ref):
    o_ref[...] = jnp.sum(x_ref[...], axis=1)


_reduce_tc = pl.pallas_call(
    _red_body,
    grid=(6,),
    in_specs=[pl.BlockSpec((1, N_TILES, 1, HALF), lambda k: (k, 0, 0, 0))],
    out_specs=pl.BlockSpec((1, 1, HALF), lambda k: (k, 0, 0)),
    out_shape=jax.ShapeDtypeStruct((6, 1, HALF), jnp.float32),
)


def _iview(u):
    return u.reshape(-1, 128, 2).transpose(0, 2, 1).reshape(-1)


def _iview3(u):
    return (u.transpose(1, 0, 2).reshape(2, -1, 128, 2)
            .transpose(0, 1, 3, 2).reshape(-1))


def kernel(pos, vel, F, C, Jp):
    del Jp
    parts = _p2g_sc(_iview(pos), _iview(vel), _iview3(F), _iview3(C))
    red = _reduce_tc(parts).reshape(2, 3, HALF)
    grid_mass = jnp.concatenate([red[0, 0], red[1, 0]])
    momx = jnp.concatenate([red[0, 1], red[1, 1]])
    momy = jnp.concatenate([red[0, 2], red[1, 2]])
    grid_mom = jnp.stack([momx, momy], axis=-1)
    return grid_mom, grid_mass


# speedup vs baseline: 1.7314x; 1.0435x over previous
import functools

import jax
import jax.numpy as jnp
from jax import lax
from jax.experimental import pallas as pl
from jax.experimental.pallas import tpu as pltpu
from jax.experimental.pallas import tpu_sc as plsc

RES = 256
DT = 1e-4
E_MOD = 10000.0
NU = 0.2
RHO = 1.0
MU0 = E_MOD / (2.0 * (1.0 + NU))
LAM0 = E_MOD * NU / ((1.0 + NU) * (1.0 - 2.0 * NU))
DX = 1.0 / RES
VOL = DX * DX * 0.5
RV = RHO * VOL
KS = -DT * VOL * (4.0 / DX ** 2)
N_P = 524288
N_TILES = 32
P_TILE = N_P // N_TILES
CH = 1024
N_CHUNK = P_TILE // CH
GROUPS = CH // 16
HALF = (RES * RES) // 2

_mesh = plsc.VectorSubcoreMesh(core_axis_name="c", subcore_axis_name="s",
                               num_cores=2, num_subcores=16)


def build_p2g(n_p, n_tiles, ch, interpret=False):
  p_tile = n_p // n_tiles
  n_chunk = p_tile // ch
  groups = ch // 16

  @functools.partial(
      pl.kernel,
      out_type=jax.ShapeDtypeStruct((6, n_tiles, 1, HALF), jnp.float32),
      mesh=_mesh,
      compiler_params=pltpu.CompilerParams(needs_layout_passes=False),
      interpret=interpret,
      scratch_types=[
          [pltpu.VMEM((2 * ch,), jnp.float32) for _ in range(6)],
          [pltpu.VMEM((2 * ch,), jnp.float32) for _ in range(6)],
          pltpu.VMEM((HALF,), jnp.float32),
          pltpu.VMEM((HALF,), jnp.float32),
          pltpu.VMEM((HALF,), jnp.float32),
          pltpu.SemaphoreType.DMA,
          pltpu.SemaphoreType.DMA,
      ],
  )
  def _p2g_sc(pxy_h, vxy_h, f_h, c_h, out_hbm,
              buf_a, buf_b, mass_v, mx_v, my_v, sem_a, sem_b):
    CH = ch
    P_TILE = p_tile
    N_CHUNK = n_chunk
    GROUPS = groups
    n2 = n_p * 2
    bufs = (buf_a, buf_b)
    sems = (sem_a, sem_b)
    wid = lax.axis_index("s") * 2 + lax.axis_index("c")
    pbase = wid * P_TILE
    zf = jnp.zeros((16,), jnp.float32)

    def srcs(k):
        w = (pbase + k * CH) * 2
        return ((pxy_h, w), (vxy_h, w), (f_h, w), (f_h, n2 + w),
                (c_h, w), (c_h, n2 + w))

    def fire(k, which):
        for (h, off), v in zip(srcs(k), bufs[which]):
            pltpu.async_copy(h.at[pl.ds(off, CH * 2)], v, sems[which])

    def wait(k, which):
        for (h, off), v in zip(srcs(k), bufs[which]):
            pltpu.make_async_copy(h.at[pl.ds(off, CH * 2)], v,
                                  sems[which]).wait()

    def make_group_body(comp_v, half):
        def group_body(g):
                base = (g // 8) * 256 + (g % 8) * 16
                r0 = base + lax.broadcasted_iota(jnp.int32, (16,), 0)
                r1 = r0 + 128
                pxy_b, vxy_b, f0_b, f1_b, c0_b, c1_b = comp_v
                px = plsc.load_gather(pxy_b, [r0])
                py = plsc.load_gather(pxy_b, [r1])
                vx = plsc.load_gather(vxy_b, [r0])
                vy = plsc.load_gather(vxy_b, [r1])
                f00 = plsc.load_gather(f0_b, [r0])
                f01 = plsc.load_gather(f0_b, [r1])
                f10 = plsc.load_gather(f1_b, [r0])
                f11 = plsc.load_gather(f1_b, [r1])
                c00 = plsc.load_gather(c0_b, [r0])
                c01 = plsc.load_gather(c0_b, [r1])
                c10 = plsc.load_gather(c1_b, [r0])
                c11 = plsc.load_gather(c1_b, [r1])

                det = f00 * f11 - f01 * f10
                rdet = 1.0 / det
                j_cl = jnp.maximum(det, 1e-8)
                nrdet = -rdet
                ft00 = f11 * rdet
                ft01 = f10 * nrdet
                ft10 = f01 * nrdet
                ft11 = f00 * rdet
                b = LAM0 * (j_cl - 1.0) - MU0
                p00 = MU0 * f00 + b * ft00
                p01 = MU0 * f01 + b * ft01
                p10 = MU0 * f10 + b * ft10
                p11 = MU0 * f11 + b * ft11
                a00 = KS * (p00 * f00 + p01 * f01) + RV * c00
                a01 = KS * (p00 * f10 + p01 * f11) + RV * c01
                a10 = KS * (p10 * f00 + p11 * f01) + RV * c10
                a11 = KS * (p10 * f10 + p11 * f11) + RV * c11

                xpx = px * float(RES) - 0.5
                xpy = py * float(RES) - 0.5
                bx = xpx.astype(jnp.int32)
                by = xpy.astype(jnp.int32)
                fxx = xpx - bx.astype(jnp.float32)
                fxy = xpy - by.astype(jnp.float32)

                tx0 = 1.5 - fxx
                tx1 = fxx - 1.0
                tx2 = fxx - 0.5
                ty0 = 1.5 - fxy
                ty1 = fxy - 1.0
                ty2 = fxy - 0.5
                wx = (0.5 * (tx0 * tx0),
                      0.75 - tx1 * tx1,
                      0.5 * (tx2 * tx2))
                wy = (0.5 * (ty0 * ty0),
                      0.75 - ty1 * ty1,
                      0.5 * (ty2 * ty2))
                wxm = tuple(w * RV for w in wx)

                mvx = RV * vx
                mvy = RV * vy
                dpx = tuple((float(i) - fxx) * DX for i in range(3))
                dpy = tuple((float(j) - fxy) * DX for j in range(3))
                xu = tuple(mvx + a00 * d for d in dpx)
                yv = tuple(mvy + a10 * d for d in dpx)
                v_j = tuple(a01 * d for d in dpy)
                s_j = tuple(a11 * d for d in dpy)

                nxs0 = tuple(
                    jnp.clip(bx + i, 0, RES - 1) << 8 for i in range(3))
                nys = tuple(
                    jnp.clip(by + j, 0, RES - 1) for j in range(3))
                if half == 0:
                    masks = tuple(n < HALF for n in nxs0)
                    nxs = nxs0
                else:
                    masks = tuple(n >= HALF for n in nxs0)
                    nxs = tuple(n - HALF for n in nxs0)

                for i in range(3):
                    m = masks[i]
                    for j in range(3):
                        idx = nxs[i] + nys[j]
                        ww = wx[i] * wy[j]
                        mass_c = wxm[i] * wy[j]
                        momx_c = ww * (xu[i] + v_j[j])
                        momy_c = ww * (yv[i] + s_j[j])
                        plsc.addupdate_scatter(mass_v, [idx], mass_c, mask=m)
                        plsc.addupdate_scatter(mx_v, [idx], momx_c, mask=m)
                        plsc.addupdate_scatter(my_v, [idx], momy_c, mask=m)
        return group_body

    for half in range(2):
        def zero_body(i, carry):
            sl = pl.ds(i * 16, 16)
            mass_v[sl] = zf
            mx_v[sl] = zf
            my_v[sl] = zf
            return carry
        lax.fori_loop(0, HALF // 16, zero_body, 0)

        fire(0, 0)

        def pair_body(t, carry):
            k0 = 2 * t
            k1 = k0 + 1
            wait(k0, 0)
            fire(k1, 1)
            plsc.parallel_loop(0, GROUPS, unroll=2)(
                make_group_body(bufs[0], half))
            wait(k1, 1)

            @pl.when(k1 + 1 < N_CHUNK)
            def _prefetch():
                fire(k1 + 1, 0)

            plsc.parallel_loop(0, GROUPS, unroll=2)(
                make_group_body(bufs[1], half))
            return carry

        lax.fori_loop(0, N_CHUNK // 2, pair_body, 0)

        pltpu.sync_copy(mass_v, out_hbm.at[3 * half + 0, wid, 0])
        pltpu.sync_copy(mx_v, out_hbm.at[3 * half + 1, wid, 0])
        pltpu.sync_copy(my_v, out_hbm.at[3 * half + 2, wid, 0])

  return _p2g_sc


_p2g_sc = build_p2g(N_P, N_TILES, CH)


def _red_body(x_ref, mass_ref, mom_ref):
    r = jnp.sum(x_ref[...].reshape(6, N_TILES, HALF), axis=1)
    mass_ref[...] = jnp.concatenate([r[0], r[3]])
    mom_ref[...] = jnp.stack([jnp.concatenate([r[1], r[4]]),
                              jnp.concatenate([r[2], r[5]])])


_reduce_tc = pl.pallas_call(
    _red_body,
    out_shape=(jax.ShapeDtypeStruct((2 * HALF,), jnp.float32),
               jax.ShapeDtypeStruct((2, 2 * HALF), jnp.float32)),
)


def _iview(u):
    return u.reshape(-1, 128, 2).transpose(0, 2, 1).reshape(-1)


def _iview3(u):
    return (u.transpose(1, 0, 2).reshape(2, -1, 128, 2)
            .transpose(0, 1, 3, 2).reshape(-1))


def kernel(pos, vel, F, C, Jp):
    del Jp
    parts = _p2g_sc(_iview(pos), _iview(vel), _iview3(F), _iview3(C))
    grid_mass, mom = _reduce_tc(parts)
    grid_mom = mom.T
    return grid_mom, grid_mass
